# Initial kernel scaffold; baseline (speedup 1.0000x reference)
import jax
import jax.numpy as jnp
from jax.experimental import pallas as pl


def kernel(x, edge_index, batch, W0, b0, g0, be0, W1, b1, g1, be1, fc1_W, fc1_b, fc2_W, fc2_b):
    raise NotImplementedError("write your pallas kernel here")


# speedup vs baseline: 10.8510x; 10.8510x over previous
import functools

import jax
import jax.numpy as jnp
from jax import lax
from jax.experimental import pallas as pl
from jax.experimental.pallas import tpu as pltpu
from jax.experimental.pallas import tpu_sc as plsc

N = 10000
E = 320000
D = 128
FC = 256
NC = 40
G = 16

NPAD = 10240
CHUNK = 128
NTILES = 32
CPT = 79
EPAD = NTILES * CPT * CHUNK
RPT = NPAD // 16

_MESH = plsc.VectorSubcoreMesh(core_axis_name="c", subcore_axis_name="s")
_f32 = jnp.float32


def _deg_body(col_hbm, ones_hbm, z8_hbm, out_hbm, idx_v, ones_v, acc_sh, sem):
    c = lax.axis_index("c")
    s = lax.axis_index("s")
    wid = c * 16 + s
    sl = pl.ds(s * RPT, RPT)
    pltpu.sync_copy(z8_hbm.at[sl], acc_sh.at[sl])
    pltpu.sync_copy(ones_hbm, ones_v)
    pltpu.sync_copy(col_hbm.at[wid], idx_v)
    plsc.subcore_barrier()

    @pl.loop(0, CPT)
    def _(j):
        pltpu.sync_copy(ones_v, acc_sh.at[idx_v.at[j]], add=True)

    plsc.subcore_barrier()
    pltpu.sync_copy(acc_sh.at[sl], out_hbm.at[c, sl])


_deg_call = pl.kernel(
    _deg_body,
    out_type=jax.ShapeDtypeStruct((2, NPAD, D), _f32),
    mesh=_MESH,
    scratch_types=[
        pltpu.VMEM((CPT, CHUNK), jnp.int32),
        pltpu.VMEM((CHUNK, D), _f32),
        pltpu.VMEM_SHARED((NPAD, D), _f32),
        pltpu.SemaphoreType.DMA,
    ],
)


def _scat_body(h_hbm, row_hbm, col_hbm, zd_hbm, out_hbm,
               rid_v, cid_v, tmp_v, acc_sh, sem):
    c = lax.axis_index("c")
    s = lax.axis_index("s")
    wid = c * 16 + s
    sl = pl.ds(s * RPT, RPT)
    pltpu.sync_copy(zd_hbm.at[sl], acc_sh.at[sl])
    pltpu.sync_copy(row_hbm.at[wid], rid_v)
    pltpu.sync_copy(col_hbm.at[wid], cid_v)
    plsc.subcore_barrier()

    @pl.loop(0, CPT)
    def _(j):
        pltpu.sync_copy(h_hbm.at[rid_v.at[j]], tmp_v)
        pltpu.sync_copy(tmp_v, acc_sh.at[cid_v.at[j]], add=True)

    plsc.subcore_barrier()
    pltpu.sync_copy(acc_sh.at[sl], out_hbm.at[c, sl])


_scat_call = pl.kernel(
    _scat_body,
    out_type=jax.ShapeDtypeStruct((2, NPAD, D), _f32),
    mesh=_MESH,
    scratch_types=[
        pltpu.VMEM((CPT, CHUNK), jnp.int32),
        pltpu.VMEM((CPT, CHUNK), jnp.int32),
        pltpu.VMEM((CHUNK, D), _f32),
        pltpu.VMEM_SHARED((NPAD, D), _f32),
        pltpu.SemaphoreType.DMA,
    ],
)


def _mm0_body(x_ref, w_ref, o_ref):
    o_ref[...] = jnp.dot(x_ref[...], w_ref[...], preferred_element_type=_f32)


_mm0_call = pl.pallas_call(
    _mm0_body, out_shape=jax.ShapeDtypeStruct((NPAD, D), _f32))


def _scale0_body(xw_ref, deg_ref, hs_ref, dis_ref):
    d = deg_ref[0][:, 0:1] + deg_ref[1][:, 0:1]
    dis = jnp.where(d > 0, lax.rsqrt(d), 0.0)
    dis_ref[...] = dis
    hs_ref[...] = xw_ref[...] * dis


_scale0_call = pl.pallas_call(
    _scale0_body,
    out_shape=(jax.ShapeDtypeStruct((NPAD, D), _f32),
               jax.ShapeDtypeStruct((NPAD, 1), _f32)))


def _mid_body(acc_ref, dis_ref, b_ref, g_ref, be_ref, w_ref, o_ref):
    dis = dis_ref[...]
    h = (acc_ref[0] + acc_ref[1]) * dis + b_ref[...]
    hr = h[0:N]
    m = jnp.mean(hr, axis=0, keepdims=True)
    v = jnp.mean((hr - m) ** 2, axis=0, keepdims=True)
    hn = (h - m) * lax.rsqrt(v + 1e-5) * g_ref[...] + be_ref[...]
    hn = jnp.maximum(hn, 0.0)
    o_ref[...] = jnp.dot(hn, w_ref[...], preferred_element_type=_f32) * dis


_mid_call = pl.pallas_call(
    _mid_body, out_shape=jax.ShapeDtypeStruct((NPAD, D), _f32))


def _fin_body(acc_ref, dis_ref, b_ref, g_ref, be_ref, batch_ref,
              w1_ref, b1_ref, w2_ref, b2_ref, o_ref):
    h = (acc_ref[0] + acc_ref[1]) * dis_ref[...] + b_ref[...]
    hr = h[0:N]
    m = jnp.mean(hr, axis=0, keepdims=True)
    v = jnp.mean((hr - m) ** 2, axis=0, keepdims=True)
    hn = (hr - m) * lax.rsqrt(v + 1e-5) * g_ref[...] + be_ref[...]
    hn = jnp.maximum(hn, 0.0)
    oh = (batch_ref[...] ==
          lax.broadcasted_iota(jnp.int32, (G, N), 0)).astype(_f32)
    pooled = jnp.dot(oh, hn, preferred_element_type=_f32)
    cnt = jnp.dot(oh, jnp.ones((N, 1), _f32), preferred_element_type=_f32)
    pooled = pooled / jnp.maximum(cnt, 1.0)
    z = jnp.maximum(
        jnp.dot(pooled, w1_ref[...], preferred_element_type=_f32) + b1_ref[...],
        0.0)
    o_ref[...] = jnp.dot(z, w2_ref[...], preferred_element_type=_f32) + b2_ref[...]


_fin_call = pl.pallas_call(
    _fin_body, out_shape=jax.ShapeDtypeStruct((G, NC), _f32))


def kernel(x, edge_index, batch, W0, b0, g0, be0, W1, b1, g1, be1,
           fc1_W, fc1_b, fc2_W, fc2_b):
    pad = jnp.full((EPAD - E,), N, jnp.int32)
    row = jnp.concatenate([edge_index[0], pad]).reshape(NTILES, CPT, CHUNK)
    col = jnp.concatenate([edge_index[1], pad]).reshape(NTILES, CPT, CHUNK)
    x_pad = jnp.pad(x, ((0, NPAD - N), (0, 0)))
    zD = jnp.zeros((NPAD, D), _f32)
    ones8 = jnp.ones((CHUNK, D), _f32)

    degp = _deg_call(col, ones8, zD)
    xw = _mm0_call(x_pad, W0)
    hs0, dis = _scale0_call(xw, degp)
    acc1 = _scat_call(hs0, row, col, zD)
    hs1 = _mid_call(acc1, dis, b0.reshape(1, D), g0.reshape(1, D),
                    be0.reshape(1, D), W1)
    acc2 = _scat_call(hs1, row, col, zD)
    out = _fin_call(acc2, dis, b1.reshape(1, D), g1.reshape(1, D),
                    be1.reshape(1, D), batch.reshape(1, N),
                    fc1_W, fc1_b.reshape(1, FC), fc2_W, fc2_b.reshape(1, NC))
    return out
